# baseline (speedup 1.0000x reference)
import functools

import jax
import jax.numpy as jnp
from jax import lax
from jax.experimental import pallas as pl
from jax.experimental.pallas import tpu as pltpu
from jax.experimental.pallas import tpu_sc as plsc

_L = 16
_NC, _NS = 2, 16
_NW = _NC * _NS
_CR = 4000


def _q_sample_sc(xf, t, nf, tab_ab, tab_mab, num_chunks):
  tlen = tab_ab.shape[0]
  n = t.shape[0]
  nk_max = (num_chunks + _NW - 1) // _NW
  mesh = plsc.VectorSubcoreMesh(
      core_axis_name="c", subcore_axis_name="s",
      num_cores=_NC, num_subcores=_NS)

  @functools.partial(
      pl.kernel,
      out_type=jax.ShapeDtypeStruct((3 * n,), jnp.float32),
      mesh=mesh,
      compiler_params=pltpu.CompilerParams(
          needs_layout_passes=False, use_tc_tiling_on_sc=False),
      scratch_types=[
          pltpu.VMEM((tlen,), jnp.float32),
          pltpu.VMEM((tlen,), jnp.float32),
          [[pltpu.VMEM((_CR,), jnp.float32)] * 6] * 2,
          [[pltpu.VMEM((_CR,), jnp.float32)] * 3] * 2,
          [pltpu.VMEM((_CR,), jnp.int32)] * 2,
          [pltpu.SemaphoreType.DMA] * 2,
          [pltpu.SemaphoreType.DMA] * 2,
      ],
  )
  def k(xf_h, t_hbm, nf_h, ab_hbm, mab_hbm, out_hbm,
        ab_v, mab_v, in_v, out_v, t_v, sem_in, sem_out):
    wid = lax.axis_index("s") * _NC + lax.axis_index("c")
    pltpu.sync_copy(ab_hbm, ab_v)
    pltpu.sync_copy(mab_hbm, mab_v)
    nk = (num_chunks - 1 - wid) // _NW + 1

    def roff_of(c):
      return (c * _NW + wid) * _CR

    def issue_in(c, b):
      roff = roff_of(c)
      for j in range(3):
        pltpu.make_async_copy(
            xf_h.at[pl.ds(j * n + roff, _CR)], in_v[b][j], sem_in[b]).start()
        pltpu.make_async_copy(
            nf_h.at[pl.ds(j * n + roff, _CR)], in_v[b][j + 3], sem_in[b]).start()
      pltpu.make_async_copy(t_hbm.at[pl.ds(roff, _CR)], t_v[b], sem_in[b]).start()

    def wait_in(c, b):
      roff = roff_of(c)
      for j in range(3):
        pltpu.make_async_copy(
            xf_h.at[pl.ds(j * n + roff, _CR)], in_v[b][j], sem_in[b]).wait()
        pltpu.make_async_copy(
            nf_h.at[pl.ds(j * n + roff, _CR)], in_v[b][j + 3], sem_in[b]).wait()
      pltpu.make_async_copy(t_hbm.at[pl.ds(roff, _CR)], t_v[b], sem_in[b]).wait()

    def issue_out(c, b):
      roff = roff_of(c)
      for j in range(3):
        pltpu.make_async_copy(
            out_v[b][j], out_hbm.at[pl.ds(j * n + roff, _CR)], sem_out[b]).start()

    def wait_out(c, b):
      roff = roff_of(c)
      for j in range(3):
        pltpu.make_async_copy(
            out_v[b][j], out_hbm.at[pl.ds(j * n + roff, _CR)], sem_out[b]).wait()

    def compute(b):
      def inner(i, c):
        sl = pl.ds(i * _L, _L)
        tv = t_v[b][sl]
        s_ab = plsc.load_gather(ab_v, [tv])
        s_mab = plsc.load_gather(mab_v, [tv])
        for j in range(3):
          out_v[b][j][sl] = s_ab * in_v[b][j][sl] + s_mab * in_v[b][j + 3][sl]
        return c

      lax.fori_loop(0, _CR // _L, inner, 0)

    issue_in(0, 0)

    def pair_body(p, carry):
      for b in range(2):
        c = 2 * p + b

        @pl.when(c < nk)
        def _():
          @pl.when(c + 1 < nk)
          def _():
            issue_in(c + 1, 1 - b)

          wait_in(c, b)

          @pl.when(c >= 2)
          def _():
            wait_out(c - 2, b)

          compute(b)
          issue_out(c, b)

      return carry

    lax.fori_loop(0, (nk_max + 1) // 2, pair_body, 0)
    wait_out(0, 0)
    wait_out(0, 1)

  return k(xf, t, nf, tab_ab, tab_mab)


_SPLIT = 496000


def kernel(x_0, t, noise, sqrt_alpha_bar, sqrt_one_minus_alpha_bar):
  n = x_0.shape[0]
  t = t.astype(jnp.int32)
  parts = []
  for lo, hi in ((0, _SPLIT), (_SPLIT, n)):
    m = hi - lo
    assert m % _CR == 0
    outf = _q_sample_sc(
        x_0[lo:hi].T.reshape(-1),
        t[lo:hi],
        noise[lo:hi].T.reshape(-1),
        sqrt_alpha_bar,
        sqrt_one_minus_alpha_bar,
        m // _CR,
    )
    parts.append(outf.reshape(3, m).T)
  out = jnp.concatenate(parts, axis=0)
  return out, noise


# speedup vs baseline: 1.0378x; 1.0378x over previous
import functools

import jax
import jax.numpy as jnp
from jax import lax
from jax.experimental import pallas as pl
from jax.experimental.pallas import tpu as pltpu
from jax.experimental.pallas import tpu_sc as plsc

_L = 16
_NC, _NS = 2, 16
_NW = _NC * _NS
_CR = 4000


def _q_sample_sc(xf, t, nf, tab_ab, tab_mab, num_chunks):
  tlen = tab_ab.shape[0]
  n = t.shape[0]
  nk_max = (num_chunks + _NW - 1) // _NW
  mesh = plsc.VectorSubcoreMesh(
      core_axis_name="c", subcore_axis_name="s",
      num_cores=_NC, num_subcores=_NS)

  @functools.partial(
      pl.kernel,
      out_type=jax.ShapeDtypeStruct((3 * n,), jnp.float32),
      mesh=mesh,
      compiler_params=pltpu.CompilerParams(
          needs_layout_passes=False, use_tc_tiling_on_sc=False),
      scratch_types=[
          pltpu.VMEM((tlen,), jnp.float32),
          pltpu.VMEM((tlen,), jnp.float32),
          [[pltpu.VMEM((_CR,), jnp.float32)] * 6] * 2,
          [[pltpu.VMEM((_CR,), jnp.float32)] * 3] * 2,
          [pltpu.VMEM((_CR,), jnp.int32)] * 2,
          [pltpu.SemaphoreType.DMA] * 2,
          [pltpu.SemaphoreType.DMA] * 2,
      ],
  )
  def k(xf_h, t_hbm, nf_h, ab_hbm, mab_hbm, out_hbm,
        ab_v, mab_v, in_v, out_v, t_v, sem_in, sem_out):
    wid = lax.axis_index("s") * _NC + lax.axis_index("c")
    pltpu.sync_copy(ab_hbm, ab_v)
    pltpu.sync_copy(mab_hbm, mab_v)
    nk = (num_chunks - 1 - wid) // _NW + 1

    def roff_of(c):
      return (c * _NW + wid) * _CR

    def issue_in(c, b):
      roff = roff_of(c)
      for j in range(3):
        pltpu.make_async_copy(
            xf_h.at[pl.ds(j * n + roff, _CR)], in_v[b][j], sem_in[b]).start()
        pltpu.make_async_copy(
            nf_h.at[pl.ds(j * n + roff, _CR)], in_v[b][j + 3], sem_in[b]).start()
      pltpu.make_async_copy(t_hbm.at[pl.ds(roff, _CR)], t_v[b], sem_in[b]).start()

    def wait_in(c, b):
      roff = roff_of(c)
      for j in range(3):
        pltpu.make_async_copy(
            xf_h.at[pl.ds(j * n + roff, _CR)], in_v[b][j], sem_in[b]).wait()
        pltpu.make_async_copy(
            nf_h.at[pl.ds(j * n + roff, _CR)], in_v[b][j + 3], sem_in[b]).wait()
      pltpu.make_async_copy(t_hbm.at[pl.ds(roff, _CR)], t_v[b], sem_in[b]).wait()

    def issue_out(c, b):
      roff = roff_of(c)
      for j in range(3):
        pltpu.make_async_copy(
            out_v[b][j], out_hbm.at[pl.ds(j * n + roff, _CR)], sem_out[b]).start()

    def wait_out(c, b):
      roff = roff_of(c)
      for j in range(3):
        pltpu.make_async_copy(
            out_v[b][j], out_hbm.at[pl.ds(j * n + roff, _CR)], sem_out[b]).wait()

    def compute(b):
      def inner(i, c):
        sl = pl.ds(i * _L, _L)
        tv = t_v[b][sl]
        s_ab = plsc.load_gather(ab_v, [tv])
        s_mab = plsc.load_gather(mab_v, [tv])
        for j in range(3):
          out_v[b][j][sl] = s_ab * in_v[b][j][sl] + s_mab * in_v[b][j + 3][sl]
        return c

      lax.fori_loop(0, _CR // _L, inner, 0)

    issue_in(0, 0)

    def pair_body(p, carry):
      for b in range(2):
        c = 2 * p + b

        @pl.when(c < nk)
        def _():
          @pl.when(c + 1 < nk)
          def _():
            issue_in(c + 1, 1 - b)

          wait_in(c, b)

          @pl.when(c >= 2)
          def _():
            wait_out(c - 2, b)

          compute(b)
          issue_out(c, b)

      return carry

    lax.fori_loop(0, (nk_max + 1) // 2, pair_body, 0)
    wait_out(0, 0)
    wait_out(0, 1)

  return k(xf, t, nf, tab_ab, tab_mab)


def kernel(x_0, t, noise, sqrt_alpha_bar, sqrt_one_minus_alpha_bar):
  n = x_0.shape[0]
  assert n % _CR == 0
  outf = _q_sample_sc(
      x_0.T.reshape(-1),
      t.astype(jnp.int32),
      noise.T.reshape(-1),
      sqrt_alpha_bar,
      sqrt_one_minus_alpha_bar,
      n // _CR,
  )
  out = outf.reshape(3, n).T
  return out, noise
